# Initial kernel scaffold; baseline (speedup 1.0000x reference)
import jax
import jax.numpy as jnp
from jax.experimental import pallas as pl


def kernel(input_seq, embedding, w_ih0, w_hh0, b_ih0, b_hh0, w_ih1, w_hh1, b_ih1, b_hh1, w_ih2, w_hh2, b_ih2, b_hh2, w_ih3, w_hh3, b_ih3, b_hh3, lin_w, lin_b):
    raise NotImplementedError("write your pallas kernel here")


# speedup vs baseline: 2.3335x; 2.3335x over previous
import jax
import jax.numpy as jnp
from jax.experimental import pallas as pl
from jax.experimental.pallas import tpu as pltpu

V = 59
E = 512
H = 256
G = 4 * H
L = 4
B = 128
T = 128
VP = 64


def _fold_emb_kernel(emb_ref, wihT_ref, out_ref):
    out_ref[...] = jnp.dot(emb_ref[...], wihT_ref[...],
                           preferred_element_type=jnp.float32)


def _gather_kernel(seq_ref, m0_ref, bias_ref, out_ref):
    seq = seq_ref[...]
    onehot = (seq == jax.lax.broadcasted_iota(jnp.int32, (1, VP), 1)
              ).astype(jnp.float32)
    out_ref[...] = jnp.dot(onehot, m0_ref[...],
                           preferred_element_type=jnp.float32) + bias_ref[...]


def _matmul_bias_kernel(x_ref, wT_ref, b_ref, out_ref):
    out_ref[...] = jnp.dot(x_ref[...], wT_ref[...],
                           preferred_element_type=jnp.float32) + b_ref[...]


def _lstm_rec_kernel(a_ref, whhT_ref, ys_ref, h_out_ref, c_out_ref,
                     h_scr, c_scr):
    t = pl.program_id(0)

    @pl.when(t == 0)
    def _init():
        h_scr[...] = jnp.zeros_like(h_scr)
        c_scr[...] = jnp.zeros_like(c_scr)

    h = h_scr[...]
    c = c_scr[...]
    gates = a_ref[0] + jnp.dot(h, whhT_ref[...],
                               preferred_element_type=jnp.float32)
    i = jax.nn.sigmoid(gates[:, 0:H])
    f = jax.nn.sigmoid(gates[:, H:2 * H])
    g = jnp.tanh(gates[:, 2 * H:3 * H])
    o = jax.nn.sigmoid(gates[:, 3 * H:4 * H])
    c_new = f * c + i * g
    h_new = o * jnp.tanh(c_new)
    h_scr[...] = h_new
    c_scr[...] = c_new
    ys_ref[0] = h_new

    @pl.when(t == T - 1)
    def _final():
        h_out_ref[...] = h_new
        c_out_ref[...] = c_new


def _proj_logsoftmax_kernel(x_ref, wT_ref, b_ref, out_ref):
    logits = jnp.dot(x_ref[...], wT_ref[...],
                     preferred_element_type=jnp.float32) + b_ref[...]
    col = jax.lax.broadcasted_iota(jnp.int32, logits.shape, 1)
    valid = col < V
    neg = jnp.float32(-1e30)
    masked = jnp.where(valid, logits, neg)
    m = jnp.max(masked, axis=1, keepdims=True)
    e = jnp.where(valid, jnp.exp(masked - m), 0.0)
    s = jnp.sum(e, axis=1, keepdims=True)
    out_ref[...] = masked - m - jnp.log(s)


def _fold_emb(emb_p, wih0T):
    return pl.pallas_call(
        _fold_emb_kernel,
        out_shape=jax.ShapeDtypeStruct((VP, G), jnp.float32),
    )(emb_p, wih0T)


def _gather_a0(seq_col, m0, bias):
    R = 1024
    grid = (T * B // R,)
    return pl.pallas_call(
        _gather_kernel,
        grid=grid,
        in_specs=[
            pl.BlockSpec((R, 1), lambda r: (r, 0)),
            pl.BlockSpec((VP, G), lambda r: (0, 0)),
            pl.BlockSpec((1, G), lambda r: (0, 0)),
        ],
        out_specs=pl.BlockSpec((R, G), lambda r: (r, 0)),
        out_shape=jax.ShapeDtypeStruct((T * B, G), jnp.float32),
    )(seq_col, m0, bias)


def _ih_matmul(x_flat, wT, bias):
    R = 2048
    grid = (T * B // R,)
    return pl.pallas_call(
        _matmul_bias_kernel,
        grid=grid,
        in_specs=[
            pl.BlockSpec((R, H), lambda r: (r, 0)),
            pl.BlockSpec((H, G), lambda r: (0, 0)),
            pl.BlockSpec((1, G), lambda r: (0, 0)),
        ],
        out_specs=pl.BlockSpec((R, G), lambda r: (r, 0)),
        out_shape=jax.ShapeDtypeStruct((T * B, G), jnp.float32),
    )(x_flat, wT, bias)


def _lstm_layer(a, whhT):
    return pl.pallas_call(
        _lstm_rec_kernel,
        grid=(T,),
        in_specs=[
            pl.BlockSpec((1, B, G), lambda t: (t, 0, 0)),
            pl.BlockSpec((H, G), lambda t: (0, 0)),
        ],
        out_specs=[
            pl.BlockSpec((1, B, H), lambda t: (t, 0, 0)),
            pl.BlockSpec((B, H), lambda t: (0, 0)),
            pl.BlockSpec((B, H), lambda t: (0, 0)),
        ],
        out_shape=[
            jax.ShapeDtypeStruct((T, B, H), jnp.float32),
            jax.ShapeDtypeStruct((B, H), jnp.float32),
            jax.ShapeDtypeStruct((B, H), jnp.float32),
        ],
        scratch_shapes=[
            pltpu.VMEM((B, H), jnp.float32),
            pltpu.VMEM((B, H), jnp.float32),
        ],
    )(a, whhT)


def _proj(x_flat, linT_p, lin_b_p):
    R = 2048
    grid = (T * B // R,)
    return pl.pallas_call(
        _proj_logsoftmax_kernel,
        grid=grid,
        in_specs=[
            pl.BlockSpec((R, H), lambda r: (r, 0)),
            pl.BlockSpec((H, VP), lambda r: (0, 0)),
            pl.BlockSpec((1, VP), lambda r: (0, 0)),
        ],
        out_specs=pl.BlockSpec((R, VP), lambda r: (r, 0)),
        out_shape=jax.ShapeDtypeStruct((T * B, VP), jnp.float32),
    )(x_flat, linT_p, lin_b_p)


def kernel(input_seq, embedding,
           w_ih0, w_hh0, b_ih0, b_hh0,
           w_ih1, w_hh1, b_ih1, b_hh1,
           w_ih2, w_hh2, b_ih2, b_hh2,
           w_ih3, w_hh3, b_ih3, b_hh3,
           lin_w, lin_b):
    seq_col = input_seq.astype(jnp.int32).T.reshape(T * B, 1)
    emb_p = jnp.pad(embedding, ((0, VP - V), (0, 0)))

    m0 = _fold_emb(emb_p, w_ih0.T)
    bias0 = (b_ih0 + b_hh0).reshape(1, G)
    a = _gather_a0(seq_col, m0, bias0)

    whhTs = [w_hh0.T, w_hh1.T, w_hh2.T, w_hh3.T]
    wihTs = [None, w_ih1.T, w_ih2.T, w_ih3.T]
    biases = [None,
              (b_ih1 + b_hh1).reshape(1, G),
              (b_ih2 + b_hh2).reshape(1, G),
              (b_ih3 + b_hh3).reshape(1, G)]

    hs, cs = [], []
    ys = None
    for l in range(L):
        if l > 0:
            a = _ih_matmul(ys.reshape(T * B, H), wihTs[l], biases[l])
        ys, hT, cT = _lstm_layer(a.reshape(T, B, G), whhTs[l])
        hs.append(hT)
        cs.append(cT)

    yb = ys.transpose(1, 0, 2).reshape(B * T, H)
    lp = _proj(yb, jnp.pad(lin_w, ((0, VP - V), (0, 0))).T,
               jnp.pad(lin_b, (0, VP - V)).reshape(1, VP))
    log_probs = lp.reshape(B, T, VP)[:, :, :V]
    return (log_probs, jnp.stack(hs, axis=0), jnp.stack(cs, axis=0))


# speedup vs baseline: 6.1886x; 2.6521x over previous
import jax
import jax.numpy as jnp
from jax.experimental import pallas as pl
from jax.experimental.pallas import tpu as pltpu

V = 59
E = 512
H = 256
G = 4 * H
L = 4
B = 128
T = 128
VP = 64
S = T + L - 1


def _fold_emb_kernel(emb_ref, wihT_ref, bias_ref, out_ref):
    out_ref[...] = jnp.dot(emb_ref[...], wihT_ref[...],
                           preferred_element_type=jnp.float32) + bias_ref[...]


def _act(gates, c):
    i = jax.nn.sigmoid(gates[:, 0:H])
    f = jax.nn.sigmoid(gates[:, H:2 * H])
    g = jnp.tanh(gates[:, 2 * H:3 * H])
    o = jax.nn.sigmoid(gates[:, 3 * H:4 * H])
    c_new = f * c + i * g
    h_new = o * jnp.tanh(c_new)
    return h_new, c_new


def _mega_kernel(seq_ref, m0b_ref, whh0T_ref, w1_ref, w2_ref, w3_ref,
                 b1_ref, b2_ref, b3_ref, linT_ref, linb_ref,
                 lp_ref, hs_ref, cs_ref,
                 h0s, c0s, h1s, c1s, h2s, c2s, h3s, c3s):
    s = pl.program_id(0)

    @pl.when(s == 0)
    def _init():
        for r in (h0s, c0s, h1s, c1s, h2s, c2s, h3s, c3s):
            r[...] = jnp.zeros_like(r)

    h0 = h0s[...]
    c0 = c0s[...]
    h1 = h1s[...]
    c1 = c1s[...]
    h2 = h2s[...]
    c2 = c2s[...]
    h3 = h3s[...]
    c3 = c3s[...]

    seq_row = seq_ref[0]
    onehotT = (seq_row == jax.lax.broadcasted_iota(jnp.int32, (VP, 1), 0)
               ).astype(jnp.float32)
    a0 = jax.lax.dot_general(onehotT, m0b_ref[...],
                             (((0,), (0,)), ((), ())),
                             preferred_element_type=jnp.float32)
    g0 = a0 + jnp.dot(h0, whh0T_ref[...], preferred_element_type=jnp.float32)
    nh0, nc0 = _act(g0, c0)

    x1 = jnp.concatenate([h0, h1], axis=1)
    g1 = jnp.dot(x1, w1_ref[...], preferred_element_type=jnp.float32) + b1_ref[...]
    nh1, nc1 = _act(g1, c1)

    x2 = jnp.concatenate([h1, h2], axis=1)
    g2 = jnp.dot(x2, w2_ref[...], preferred_element_type=jnp.float32) + b2_ref[...]
    nh2, nc2 = _act(g2, c2)

    x3 = jnp.concatenate([h2, h3], axis=1)
    g3 = jnp.dot(x3, w3_ref[...], preferred_element_type=jnp.float32) + b3_ref[...]
    nh3, nc3 = _act(g3, c3)

    logits = jnp.dot(nh3, linT_ref[...],
                     preferred_element_type=jnp.float32) + linb_ref[...]
    col = jax.lax.broadcasted_iota(jnp.int32, logits.shape, 1)
    valid = col < V
    masked = jnp.where(valid, logits, jnp.float32(-1e30))
    m = jnp.max(masked, axis=1, keepdims=True)
    e = jnp.where(valid, jnp.exp(masked - m), 0.0)
    lp_ref[0] = masked - m - jnp.log(jnp.sum(e, axis=1, keepdims=True))

    @pl.when(s < T)
    def _u0():
        h0s[...] = nh0
        c0s[...] = nc0

    @pl.when((s >= 1) & (s < T + 1))
    def _u1():
        h1s[...] = nh1
        c1s[...] = nc1

    @pl.when((s >= 2) & (s < T + 2))
    def _u2():
        h2s[...] = nh2
        c2s[...] = nc2

    @pl.when(s >= 3)
    def _u3():
        h3s[...] = nh3
        c3s[...] = nc3

    @pl.when(s == S - 1)
    def _cap():
        hs_ref[0] = h0s[...]
        hs_ref[1] = h1s[...]
        hs_ref[2] = h2s[...]
        hs_ref[3] = nh3
        cs_ref[0] = c0s[...]
        cs_ref[1] = c1s[...]
        cs_ref[2] = c2s[...]
        cs_ref[3] = nc3


def kernel(input_seq, embedding,
           w_ih0, w_hh0, b_ih0, b_hh0,
           w_ih1, w_hh1, b_ih1, b_hh1,
           w_ih2, w_hh2, b_ih2, b_hh2,
           w_ih3, w_hh3, b_ih3, b_hh3,
           lin_w, lin_b):
    seq = input_seq.astype(jnp.int32).T.reshape(T, 1, B)
    emb_p = jnp.pad(embedding, ((0, VP - V), (0, 0)))
    bias0 = (b_ih0 + b_hh0).reshape(1, G)

    m0b = pl.pallas_call(
        _fold_emb_kernel,
        out_shape=jax.ShapeDtypeStruct((VP, G), jnp.float32),
    )(emb_p, w_ih0.T, bias0)

    w1 = jnp.concatenate([w_ih1.T, w_hh1.T], axis=0)
    w2 = jnp.concatenate([w_ih2.T, w_hh2.T], axis=0)
    w3 = jnp.concatenate([w_ih3.T, w_hh3.T], axis=0)
    b1 = (b_ih1 + b_hh1).reshape(1, G)
    b2 = (b_ih2 + b_hh2).reshape(1, G)
    b3 = (b_ih3 + b_hh3).reshape(1, G)
    linT = jnp.pad(lin_w, ((0, VP - V), (0, 0))).T
    linb = jnp.pad(lin_b, (0, VP - V)).reshape(1, VP)

    full = lambda shape: pl.BlockSpec(shape, lambda s: tuple(0 for _ in shape))
    lp, hs, cs = pl.pallas_call(
        _mega_kernel,
        grid=(S,),
        in_specs=[
            pl.BlockSpec((1, 1, B),
                         lambda s: (jnp.minimum(s, T - 1), 0, 0)),
            full((VP, G)),
            full((H, G)),
            full((2 * H, G)),
            full((2 * H, G)),
            full((2 * H, G)),
            full((1, G)),
            full((1, G)),
            full((1, G)),
            full((H, VP)),
            full((1, VP)),
        ],
        out_specs=[
            pl.BlockSpec((1, B, VP), lambda s: (jnp.maximum(s - (L - 1), 0), 0, 0)),
            pl.BlockSpec((L, B, H), lambda s: (0, 0, 0)),
            pl.BlockSpec((L, B, H), lambda s: (0, 0, 0)),
        ],
        out_shape=[
            jax.ShapeDtypeStruct((T, B, VP), jnp.float32),
            jax.ShapeDtypeStruct((L, B, H), jnp.float32),
            jax.ShapeDtypeStruct((L, B, H), jnp.float32),
        ],
        scratch_shapes=[pltpu.VMEM((B, H), jnp.float32) for _ in range(2 * L)],
    )(seq, m0b, w_hh0.T, w1, w2, w3, b1, b2, b3, linT, linb)

    log_probs = lp.transpose(1, 0, 2)[:, :, :V]
    return (log_probs, hs, cs)


# speedup vs baseline: 6.8400x; 1.1053x over previous
import jax
import jax.numpy as jnp
from jax.experimental import pallas as pl
from jax.experimental.pallas import tpu as pltpu

V = 59
E = 512
H = 256
G = 4 * H
L = 4
B = 128
T = 128
VP = 64
S = T + L - 1


def _fold_emb_kernel(emb_ref, wihT_ref, bias_ref, out_ref):
    out_ref[...] = (jnp.dot(emb_ref[...], wihT_ref[...],
                            preferred_element_type=jnp.float32)
                    + bias_ref[...]).astype(jnp.bfloat16)


def _sig(x):
    return 0.5 * jnp.tanh(0.5 * x) + 0.5


def _act(gates, c):
    i = _sig(gates[:, 0:H])
    f = _sig(gates[:, H:2 * H])
    g = jnp.tanh(gates[:, 2 * H:3 * H])
    o = _sig(gates[:, 3 * H:4 * H])
    c_new = f * c + i * g
    h_new = o * jnp.tanh(c_new)
    return h_new, c_new


def _mega_kernel(seq_ref, m0b_ref, whh0T_ref, w1_ref, w2_ref, w3_ref,
                 b1_ref, b2_ref, b3_ref, linT_ref, linb_ref,
                 lp_ref, hs_ref, cs_ref,
                 h0s, c0s, h1s, c1s, h2s, c2s, h3s, c3s):
    s = pl.program_id(0)

    @pl.when(s == 0)
    def _init():
        for r in (h0s, c0s, h1s, c1s, h2s, c2s, h3s, c3s):
            r[...] = jnp.zeros_like(r)

    h0 = h0s[...]
    c0 = c0s[...]
    h1 = h1s[...]
    c1 = c1s[...]
    h2 = h2s[...]
    c2 = c2s[...]
    h3 = h3s[...]
    c3 = c3s[...]

    bf = jnp.bfloat16
    h0b = h0.astype(bf)
    h1b = h1.astype(bf)
    h2b = h2.astype(bf)
    h3b = h3.astype(bf)

    seq_row = seq_ref[0]
    onehotT = (seq_row == jax.lax.broadcasted_iota(jnp.int32, (VP, 1), 0)
               ).astype(bf)
    a0 = jax.lax.dot_general(onehotT, m0b_ref[...],
                             (((0,), (0,)), ((), ())),
                             preferred_element_type=jnp.float32)
    g0 = a0 + jnp.dot(h0b, whh0T_ref[...], preferred_element_type=jnp.float32)
    nh0, nc0 = _act(g0, c0)

    x1 = jnp.concatenate([h0b, h1b], axis=1)
    g1 = jnp.dot(x1, w1_ref[...], preferred_element_type=jnp.float32) + b1_ref[...]
    nh1, nc1 = _act(g1, c1)

    x2 = jnp.concatenate([h1b, h2b], axis=1)
    g2 = jnp.dot(x2, w2_ref[...], preferred_element_type=jnp.float32) + b2_ref[...]
    nh2, nc2 = _act(g2, c2)

    x3 = jnp.concatenate([h2b, h3b], axis=1)
    g3 = jnp.dot(x3, w3_ref[...], preferred_element_type=jnp.float32) + b3_ref[...]
    nh3, nc3 = _act(g3, c3)

    logits = jnp.dot(nh3.astype(bf), linT_ref[...],
                     preferred_element_type=jnp.float32) + linb_ref[...]
    col = jax.lax.broadcasted_iota(jnp.int32, logits.shape, 1)
    valid = col < V
    masked = jnp.where(valid, logits, jnp.float32(-1e30))
    m = jnp.max(masked, axis=1, keepdims=True)
    e = jnp.where(valid, jnp.exp(masked - m), 0.0)
    lp_ref[0] = masked - m - jnp.log(jnp.sum(e, axis=1, keepdims=True))

    @pl.when(s < T)
    def _u0():
        h0s[...] = nh0
        c0s[...] = nc0

    @pl.when((s >= 1) & (s < T + 1))
    def _u1():
        h1s[...] = nh1
        c1s[...] = nc1

    @pl.when((s >= 2) & (s < T + 2))
    def _u2():
        h2s[...] = nh2
        c2s[...] = nc2

    @pl.when(s >= 3)
    def _u3():
        h3s[...] = nh3
        c3s[...] = nc3

    @pl.when(s == S - 1)
    def _cap():
        hs_ref[0] = h0s[...]
        hs_ref[1] = h1s[...]
        hs_ref[2] = h2s[...]
        hs_ref[3] = nh3
        cs_ref[0] = c0s[...]
        cs_ref[1] = c1s[...]
        cs_ref[2] = c2s[...]
        cs_ref[3] = nc3


def kernel(input_seq, embedding,
           w_ih0, w_hh0, b_ih0, b_hh0,
           w_ih1, w_hh1, b_ih1, b_hh1,
           w_ih2, w_hh2, b_ih2, b_hh2,
           w_ih3, w_hh3, b_ih3, b_hh3,
           lin_w, lin_b):
    seq = input_seq.astype(jnp.int32).T.reshape(T, 1, B)
    emb_p = jnp.pad(embedding, ((0, VP - V), (0, 0)))
    bias0 = (b_ih0 + b_hh0).reshape(1, G)

    m0b = pl.pallas_call(
        _fold_emb_kernel,
        out_shape=jax.ShapeDtypeStruct((VP, G), jnp.bfloat16),
    )(emb_p, w_ih0.T, bias0)

    bf = jnp.bfloat16
    w1 = jnp.concatenate([w_ih1.T, w_hh1.T], axis=0).astype(bf)
    w2 = jnp.concatenate([w_ih2.T, w_hh2.T], axis=0).astype(bf)
    w3 = jnp.concatenate([w_ih3.T, w_hh3.T], axis=0).astype(bf)
    whh0T = w_hh0.T.astype(bf)
    b1 = (b_ih1 + b_hh1).reshape(1, G)
    b2 = (b_ih2 + b_hh2).reshape(1, G)
    b3 = (b_ih3 + b_hh3).reshape(1, G)
    linT = jnp.pad(lin_w, ((0, VP - V), (0, 0))).T.astype(bf)
    linb = jnp.pad(lin_b, (0, VP - V)).reshape(1, VP)

    full = lambda shape: pl.BlockSpec(shape, lambda s: tuple(0 for _ in shape))
    lp, hs, cs = pl.pallas_call(
        _mega_kernel,
        grid=(S,),
        in_specs=[
            pl.BlockSpec((1, 1, B),
                         lambda s: (jnp.minimum(s, T - 1), 0, 0)),
            full((VP, G)),
            full((H, G)),
            full((2 * H, G)),
            full((2 * H, G)),
            full((2 * H, G)),
            full((1, G)),
            full((1, G)),
            full((1, G)),
            full((H, VP)),
            full((1, VP)),
        ],
        out_specs=[
            pl.BlockSpec((1, B, VP), lambda s: (jnp.maximum(s - (L - 1), 0), 0, 0)),
            pl.BlockSpec((L, B, H), lambda s: (0, 0, 0)),
            pl.BlockSpec((L, B, H), lambda s: (0, 0, 0)),
        ],
        out_shape=[
            jax.ShapeDtypeStruct((T, B, VP), jnp.float32),
            jax.ShapeDtypeStruct((L, B, H), jnp.float32),
            jax.ShapeDtypeStruct((L, B, H), jnp.float32),
        ],
        scratch_shapes=[pltpu.VMEM((B, H), jnp.float32) for _ in range(2 * L)],
    )(seq, m0b, whh0T, w1, w2, w3, b1, b2, b3, linT, linb)

    log_probs = lp.transpose(1, 0, 2)[:, :, :V]
    return (log_probs, hs, cs)


# speedup vs baseline: 7.4127x; 1.0837x over previous
import jax
import jax.numpy as jnp
from jax.experimental import pallas as pl
from jax.experimental.pallas import tpu as pltpu

V = 59
E = 512
H = 256
G = 4 * H
L = 4
B = 128
T = 128
VP = 64
U = T // 2 + 2


def _fold_emb_kernel(emb_ref, wihT_ref, bias_ref, out_ref):
    out_ref[...] = (jnp.dot(emb_ref[...], wihT_ref[...],
                            preferred_element_type=jnp.float32)
                    + bias_ref[...]).astype(jnp.bfloat16)


def _act(gates, c):
    ti = jnp.tanh(gates[:, 0:H])
    tf = jnp.tanh(gates[:, H:2 * H])
    g = jnp.tanh(gates[:, 2 * H:3 * H])
    to = jnp.tanh(gates[:, 3 * H:4 * H])
    c_new = 0.5 * ((1.0 + tf) * c + (1.0 + ti) * g)
    h_new = (0.5 * (1.0 + to)) * jnp.tanh(c_new)
    return h_new, c_new


def _mega_kernel(seq_ref, m0b_ref, whh0T_ref, w1_ref, w2_ref, w3_ref,
                 b1_ref, b2_ref, b3_ref, linT_ref, linb_ref,
                 lp_ref, hs_ref, cs_ref,
                 h0s, c0s, h1s, c1s, h2s, c2s, h3s, c3s):
    u = pl.program_id(0)
    bf = jnp.bfloat16

    @pl.when(u == 0)
    def _init():
        for r in (h0s, c0s, h1s, c1s, h2s, c2s, h3s, c3s):
            r[...] = jnp.zeros_like(r)

    def substep(k, h0, c0, h1, c1, h2, c2, h3, c3):
        h0b = h0.astype(bf)
        h1b = h1.astype(bf)
        h2b = h2.astype(bf)
        h3b = h3.astype(bf)

        seq_row = seq_ref[0, k:k + 1]
        onehotT = (seq_row == jax.lax.broadcasted_iota(jnp.int32, (VP, 1), 0)
                   ).astype(bf)
        a0 = jax.lax.dot_general(onehotT, m0b_ref[...],
                                 (((0,), (0,)), ((), ())),
                                 preferred_element_type=jnp.float32)
        g0 = a0 + jnp.dot(h0b, whh0T_ref[...],
                          preferred_element_type=jnp.float32)
        nh0, nc0 = _act(g0, c0)

        x1 = jnp.concatenate([h0b, h1b], axis=1)
        g1 = jnp.dot(x1, w1_ref[...],
                     preferred_element_type=jnp.float32) + b1_ref[...]
        nh1, nc1 = _act(g1, c1)

        x2 = jnp.concatenate([h1b, h2b], axis=1)
        g2 = jnp.dot(x2, w2_ref[...],
                     preferred_element_type=jnp.float32) + b2_ref[...]
        nh2, nc2 = _act(g2, c2)

        x3 = jnp.concatenate([h2b, h3b], axis=1)
        g3 = jnp.dot(x3, w3_ref[...],
                     preferred_element_type=jnp.float32) + b3_ref[...]
        nh3, nc3 = _act(g3, c3)

        logits = jnp.dot(nh3.astype(bf), linT_ref[...],
                         preferred_element_type=jnp.float32) + linb_ref[...]
        col = jax.lax.broadcasted_iota(jnp.int32, logits.shape, 1)
        valid = col < V
        masked = jnp.where(valid, logits, jnp.float32(-1e30))
        m = jnp.max(masked, axis=1, keepdims=True)
        e = jnp.where(valid, jnp.exp(masked - m), 0.0)
        lp_ref[0, k] = masked - m - jnp.log(jnp.sum(e, axis=1, keepdims=True))
        return nh0, nc0, nh1, nc1, nh2, nc2, nh3, nc3

    st = (h0s[...], c0s[...], h1s[...], c1s[...],
          h2s[...], c2s[...], h3s[...], c3s[...])
    mid = substep(0, *st)
    nh0, nc0, nh1, nc1, nh2, nc2, nh3, nc3 = substep(1, *mid)

    @pl.when(u <= T // 2 - 1)
    def _u0():
        h0s[...] = nh0
        c0s[...] = nc0

    @pl.when(u <= T // 2 - 1)
    def _u1():
        h1s[...] = nh1
        c1s[...] = nc1

    @pl.when(u == T // 2)
    def _u1f():
        h1s[...] = mid[2]
        c1s[...] = mid[3]

    @pl.when((u >= 1) & (u <= T // 2))
    def _u2():
        h2s[...] = nh2
        c2s[...] = nc2

    @pl.when((u >= 1) & (u <= T // 2))
    def _u3():
        h3s[...] = nh3
        c3s[...] = nc3

    @pl.when(u == U - 1)
    def _cap():
        hs_ref[0] = h0s[...]
        hs_ref[1] = h1s[...]
        hs_ref[2] = h2s[...]
        hs_ref[3] = mid[6]
        cs_ref[0] = c0s[...]
        cs_ref[1] = c1s[...]
        cs_ref[2] = c2s[...]
        cs_ref[3] = mid[7]


def kernel(input_seq, embedding,
           w_ih0, w_hh0, b_ih0, b_hh0,
           w_ih1, w_hh1, b_ih1, b_hh1,
           w_ih2, w_hh2, b_ih2, b_hh2,
           w_ih3, w_hh3, b_ih3, b_hh3,
           lin_w, lin_b):
    seq = input_seq.astype(jnp.int32).T.reshape(T // 2, 2, B)
    emb_p = jnp.pad(embedding, ((0, VP - V), (0, 0)))

    colscale = jnp.concatenate([
        jnp.full((2 * H,), 0.5, jnp.float32),
        jnp.ones((H,), jnp.float32),
        jnp.full((H,), 0.5, jnp.float32)]).reshape(1, G)
    bias0 = (b_ih0 + b_hh0).reshape(1, G) * colscale

    m0b = pl.pallas_call(
        _fold_emb_kernel,
        out_shape=jax.ShapeDtypeStruct((VP, G), jnp.bfloat16),
    )(emb_p, w_ih0.T * colscale, bias0)

    bf = jnp.bfloat16
    w1 = (jnp.concatenate([w_ih1.T, w_hh1.T], axis=0) * colscale).astype(bf)
    w2 = (jnp.concatenate([w_ih2.T, w_hh2.T], axis=0) * colscale).astype(bf)
    w3 = (jnp.concatenate([w_ih3.T, w_hh3.T], axis=0) * colscale).astype(bf)
    whh0T = (w_hh0.T * colscale).astype(bf)
    b1 = (b_ih1 + b_hh1).reshape(1, G) * colscale
    b2 = (b_ih2 + b_hh2).reshape(1, G) * colscale
    b3 = (b_ih3 + b_hh3).reshape(1, G) * colscale
    linT = jnp.pad(lin_w, ((0, VP - V), (0, 0))).T.astype(bf)
    linb = jnp.pad(lin_b, (0, VP - V)).reshape(1, VP)

    full = lambda shape: pl.BlockSpec(shape, lambda u: tuple(0 for _ in shape))
    lp, hs, cs = pl.pallas_call(
        _mega_kernel,
        grid=(U,),
        in_specs=[
            pl.BlockSpec((1, 2, B),
                         lambda u: (jnp.minimum(u, T // 2 - 1), 0, 0)),
            full((VP, G)),
            full((H, G)),
            full((2 * H, G)),
            full((2 * H, G)),
            full((2 * H, G)),
            full((1, G)),
            full((1, G)),
            full((1, G)),
            full((H, VP)),
            full((1, VP)),
        ],
        out_specs=[
            pl.BlockSpec((1, 2, B, VP),
                         lambda u: (jnp.maximum(u - 1, 0), 0, 0, 0)),
            pl.BlockSpec((L, B, H), lambda u: (0, 0, 0)),
            pl.BlockSpec((L, B, H), lambda u: (0, 0, 0)),
        ],
        out_shape=[
            jax.ShapeDtypeStruct((T // 2 + 1, 2, B, VP), jnp.float32),
            jax.ShapeDtypeStruct((L, B, H), jnp.float32),
            jax.ShapeDtypeStruct((L, B, H), jnp.float32),
        ],
        scratch_shapes=[pltpu.VMEM((B, H), jnp.float32) for _ in range(2 * L)],
    )(seq, m0b, whh0T, w1, w2, w3, b1, b2, b3, linT, linb)

    log_probs = lp.reshape(T + 2, B, VP)[1:T + 1].transpose(1, 0, 2)[:, :, :V]
    return (log_probs, hs, cs)
